# baseline (speedup 1.0000x reference)
import functools

import jax
import jax.numpy as jnp
from jax import lax
from jax.experimental import pallas as pl
from jax.experimental.pallas import tpu as pltpu
from jax.experimental.pallas import tpu_sc as plsc

B = 16384
EMB = 64
EMB_HALF = 32
HID = 128
N_USERS = 1000000
N_ITEMS = 100000

NC = 2
NS = 16
NW = NC * NS
BPW = B // NW
CHUNK = 128
NCH = BPW // CHUNK

BLK = 2048
FLW = 8192
HALF = FLW // 2
QTR = FLW // 4


def _flatten_body(tT, eye, o):
    xb = tT[...].astype(jnp.bfloat16)
    vT = lax.dot_general(
        xb, eye[...], (((0,), (0,)), ((), ())),
        preferred_element_type=jnp.float32,
    )
    w = pltpu.bitcast(vT.astype(jnp.bfloat16), jnp.int32)
    o[:, 0:EMB] = w[0:QTR]
    o[:, EMB:128] = w[QTR : 2 * QTR]


@functools.cache
def _make_flatten(n_rows):
    grid = -(-n_rows // FLW)
    return pl.pallas_call(
        _flatten_body,
        grid=(grid,),
        in_specs=[
            pl.BlockSpec((EMB, FLW), lambda i: (0, i)),
            pl.BlockSpec((EMB, EMB), lambda i: (0, 0)),
        ],
        out_specs=pl.BlockSpec((QTR, 128), lambda i: (i, 0)),
        out_shape=jax.ShapeDtypeStruct((grid * QTR, 128), jnp.int32),
    )


def _carrier_slot(r):
    q = r // 2
    l2 = q % (FLW // 2)
    return (q // (FLW // 2)) * (FLW // 2) + 2 * (l2 % QTR) + l2 // QTR


def _mesh():
    return plsc.VectorSubcoreMesh(
        core_axis_name="c", subcore_axis_name="s", num_cores=NC, num_subcores=NS
    )


def _gather_ibc_body(
    i_idx, b_idx, c_idx,
    i_tab, b_tab, c_tab,
    i_out, b_out, c_out,
    i_iv, b_iv, c_iv,
    i_rows, b_rows, c_rows,
    sem,
):
    wid = lax.axis_index("s") * NC + lax.axis_index("c")
    base = wid * BPW

    pltpu.sync_copy(i_idx.at[wid], i_iv)
    pltpu.sync_copy(b_idx.at[wid], b_iv)
    pltpu.sync_copy(c_idx.at[wid], c_iv)

    copies = []
    for j in range(NCH):
        sl = pl.ds(j * CHUNK, CHUNK)
        copies.append(pltpu.async_copy(i_tab.at[i_iv.at[j]], i_rows.at[sl], sem))
        copies.append(pltpu.async_copy(b_tab.at[b_iv.at[j]], b_rows.at[sl], sem))
        copies.append(pltpu.async_copy(c_tab.at[c_iv.at[j]], c_rows.at[sl], sem))
    for cp in copies:
        cp.wait()

    out_sl = pl.ds(base, BPW)
    pltpu.sync_copy(i_rows, i_out.at[out_sl])
    pltpu.sync_copy(b_rows, b_out.at[out_sl])
    pltpu.sync_copy(c_rows, c_out.at[out_sl])


def _gather_u_body(u_idx, u_tab, u_out, u_iv, u_rows, sem):
    wid = lax.axis_index("s") * NC + lax.axis_index("c")
    base = wid * BPW

    pltpu.sync_copy(u_idx.at[wid], u_iv)
    copies = []
    for j in range(NCH):
        sl = pl.ds(j * CHUNK, CHUNK)
        copies.append(pltpu.async_copy(u_tab.at[u_iv.at[j]], u_rows.at[sl], sem))
    for cp in copies:
        cp.wait()
    pltpu.sync_copy(u_rows, u_out.at[pl.ds(base, BPW)])


@functools.cache
def _make_gather_ibc():
    return pl.kernel(
        _gather_ibc_body,
        out_type=(
            jax.ShapeDtypeStruct((B, EMB), jnp.int32),
            jax.ShapeDtypeStruct((B, EMB_HALF), jnp.float32),
            jax.ShapeDtypeStruct((B, EMB_HALF), jnp.float32),
        ),
        mesh=_mesh(),
        scratch_types=[
            pltpu.VMEM((NCH, CHUNK), jnp.int32),
            pltpu.VMEM((NCH, CHUNK), jnp.int32),
            pltpu.VMEM((NCH, CHUNK), jnp.int32),
            pltpu.VMEM((BPW, EMB), jnp.int32),
            pltpu.VMEM((BPW, EMB_HALF), jnp.float32),
            pltpu.VMEM((BPW, EMB_HALF), jnp.float32),
            pltpu.SemaphoreType.DMA,
        ],
        compiler_params=pltpu.CompilerParams(use_tc_tiling_on_sc=False),
    )


@functools.cache
def _make_gather_u():
    return pl.kernel(
        _gather_u_body,
        out_type=jax.ShapeDtypeStruct((B, EMB), jnp.int32),
        mesh=_mesh(),
        scratch_types=[
            pltpu.VMEM((NCH, CHUNK), jnp.int32),
            pltpu.VMEM((BPW, EMB), jnp.int32),
            pltpu.SemaphoreType.DMA,
        ],
        compiler_params=pltpu.CompilerParams(use_tc_tiling_on_sc=False),
    )


def _unpack(w):
    wu = lax.bitcast_convert_type(w, jnp.uint32)
    lo = lax.bitcast_convert_type((wu & 0xFFFF).astype(jnp.uint16), jnp.bfloat16)
    hi = lax.bitcast_convert_type((wu >> 16).astype(jnp.uint16), jnp.bfloat16)
    return lo, hi


def _mlp_body(u, i, b, c, p, upar, ipar, w1u, w1i, w1b, w1c, w1p, b1, w2, b2, o):
    f32 = jnp.float32
    ulo, uhi = _unpack(u[...])
    ilo, ihi = _unpack(i[...])
    ub = jnp.where(upar[...] != 0, uhi, ulo)
    ib = jnp.where(ipar[...] != 0, ihi, ilo)
    h = jnp.dot(ub, w1u[...], preferred_element_type=f32)
    h = h + jnp.dot(ib, w1i[...], preferred_element_type=f32)
    h = h + jnp.dot(b[...].astype(jnp.bfloat16), w1b[...], preferred_element_type=f32)
    h = h + jnp.dot(c[...].astype(jnp.bfloat16), w1c[...], preferred_element_type=f32)
    h = h + p[...] * w1p[...]
    h = h + b1[...]
    h = jnp.maximum(h, 0.0)
    o[...] = jnp.dot(h, w2[...], preferred_element_type=f32) + b2[...]


def _make_mlp(interpret=False):
    rep = lambda idx: (0, 0)
    row = lambda idx: (idx, 0)
    return pl.pallas_call(
        _mlp_body,
        grid=(B // BLK,),
        in_specs=[
            pl.BlockSpec((BLK, EMB), row),
            pl.BlockSpec((BLK, EMB), row),
            pl.BlockSpec((BLK, EMB_HALF), row),
            pl.BlockSpec((BLK, EMB_HALF), row),
            pl.BlockSpec((BLK, 1), row),
            pl.BlockSpec((BLK, 1), row),
            pl.BlockSpec((BLK, 1), row),
            pl.BlockSpec((EMB, HID), rep),
            pl.BlockSpec((EMB, HID), rep),
            pl.BlockSpec((EMB_HALF, HID), rep),
            pl.BlockSpec((EMB_HALF, HID), rep),
            pl.BlockSpec((1, HID), rep),
            pl.BlockSpec((1, HID), rep),
            pl.BlockSpec((HID, 1), rep),
            pl.BlockSpec((1, 1), rep),
        ],
        out_specs=pl.BlockSpec((BLK, 1), row),
        out_shape=jax.ShapeDtypeStruct((B, 1), jnp.float32),
        interpret=interpret,
    )


_mlp = _make_mlp()


def kernel(user, item, brand, category, price, user_emb, item_emb, brand_emb,
           cat_emb, W1, b1, W2, b2):
    u_idx = _carrier_slot(user.astype(jnp.int32)).reshape(NW, NCH, CHUNK)
    i_idx = _carrier_slot(item.astype(jnp.int32)).reshape(NW, NCH, CHUNK)
    b_idx = brand.astype(jnp.int32).reshape(NW, NCH, CHUNK)
    c_idx = category.astype(jnp.int32).reshape(NW, NCH, CHUNK)

    eye = jnp.eye(EMB, dtype=jnp.bfloat16)
    i_carr = _make_flatten(N_ITEMS)(item_emb.T, eye)
    i_tab = i_carr.reshape(i_carr.shape[0] * 2, EMB)
    i, b, c = _make_gather_ibc()(i_idx, b_idx, c_idx, i_tab, brand_emb, cat_emb)

    u_carr = _make_flatten(N_USERS)(user_emb.T, eye)
    u_tab = u_carr.reshape(u_carr.shape[0] * 2, EMB)
    u = _make_gather_u()(u_idx, u_tab)

    bf = jnp.bfloat16
    w1u = W1[:EMB].astype(bf)
    w1i = W1[EMB : 2 * EMB].astype(bf)
    w1b = W1[2 * EMB : 2 * EMB + EMB_HALF].astype(bf)
    w1c = W1[2 * EMB + EMB_HALF : 3 * EMB].astype(bf)
    w1p = W1[3 * EMB :]

    upar = (user.astype(jnp.int32) & 1)[:, None]
    ipar = (item.astype(jnp.int32) & 1)[:, None]
    out = _mlp(
        u, i, b, c, price[:, None], upar, ipar,
        w1u, w1i, w1b, w1c, w1p,
        b1[None, :], W2, b2[None, :],
    )
    return out[:, 0]


# speedup vs baseline: 1.1546x; 1.1546x over previous
import functools

import jax
import jax.numpy as jnp
from jax import lax
from jax.experimental import pallas as pl
from jax.experimental.pallas import tpu as pltpu
from jax.experimental.pallas import tpu_sc as plsc

B = 16384
EMB = 64
EMB_HALF = 32
HID = 128
N_USERS = 1000000
N_ITEMS = 100000

NC = 2
NS = 16
NW = NC * NS
BPW = B // NW
CHUNK = 128
NCH = BPW // CHUNK

BLK = 2048
FLW = 16384
HALF = FLW // 2
QTR = FLW // 4


def _flatten_body(tT, eye, o):
    xb = tT[...].astype(jnp.bfloat16)
    vT = lax.dot_general(
        xb, eye[...], (((0,), (0,)), ((), ())),
        preferred_element_type=jnp.float32,
    )
    w = pltpu.bitcast(vT.astype(jnp.bfloat16), jnp.int32)
    o[:, 0:EMB] = w[0:QTR]
    o[:, EMB:128] = w[QTR : 2 * QTR]


@functools.cache
def _make_flatten(n_rows):
    grid = -(-n_rows // FLW)
    return pl.pallas_call(
        _flatten_body,
        grid=(grid,),
        in_specs=[
            pl.BlockSpec((EMB, FLW), lambda i: (0, i)),
            pl.BlockSpec((EMB, EMB), lambda i: (0, 0)),
        ],
        out_specs=pl.BlockSpec((QTR, 128), lambda i: (i, 0)),
        out_shape=jax.ShapeDtypeStruct((grid * QTR, 128), jnp.int32),
    )


def _carrier_slot(r):
    q = r // 2
    l2 = q % (FLW // 2)
    return (q // (FLW // 2)) * (FLW // 2) + 2 * (l2 % QTR) + l2 // QTR


def _mesh():
    return plsc.VectorSubcoreMesh(
        core_axis_name="c", subcore_axis_name="s", num_cores=NC, num_subcores=NS
    )


def _gather_ibc_body(
    i_idx, b_idx, c_idx,
    i_tab, b_tab, c_tab,
    i_out, b_out, c_out,
    i_iv, b_iv, c_iv,
    i_rows, b_rows, c_rows,
    sem,
):
    wid = lax.axis_index("s") * NC + lax.axis_index("c")
    base = wid * BPW

    pltpu.sync_copy(i_idx.at[wid], i_iv)
    pltpu.sync_copy(b_idx.at[wid], b_iv)
    pltpu.sync_copy(c_idx.at[wid], c_iv)

    copies = []
    for j in range(NCH):
        sl = pl.ds(j * CHUNK, CHUNK)
        copies.append(pltpu.async_copy(i_tab.at[i_iv.at[j]], i_rows.at[sl], sem))
        copies.append(pltpu.async_copy(b_tab.at[b_iv.at[j]], b_rows.at[sl], sem))
        copies.append(pltpu.async_copy(c_tab.at[c_iv.at[j]], c_rows.at[sl], sem))
    for cp in copies:
        cp.wait()

    out_sl = pl.ds(base, BPW)
    pltpu.sync_copy(i_rows, i_out.at[out_sl])
    pltpu.sync_copy(b_rows, b_out.at[out_sl])
    pltpu.sync_copy(c_rows, c_out.at[out_sl])


def _gather_u_body(u_idx, u_tab, u_out, u_iv, u_rows, sem):
    wid = lax.axis_index("s") * NC + lax.axis_index("c")
    base = wid * BPW

    pltpu.sync_copy(u_idx.at[wid], u_iv)
    copies = []
    for j in range(NCH):
        sl = pl.ds(j * CHUNK, CHUNK)
        copies.append(pltpu.async_copy(u_tab.at[u_iv.at[j]], u_rows.at[sl], sem))
    for cp in copies:
        cp.wait()
    pltpu.sync_copy(u_rows, u_out.at[pl.ds(base, BPW)])


@functools.cache
def _make_gather_ibc():
    return pl.kernel(
        _gather_ibc_body,
        out_type=(
            jax.ShapeDtypeStruct((B, EMB), jnp.int32),
            jax.ShapeDtypeStruct((B, EMB_HALF), jnp.float32),
            jax.ShapeDtypeStruct((B, EMB_HALF), jnp.float32),
        ),
        mesh=_mesh(),
        scratch_types=[
            pltpu.VMEM((NCH, CHUNK), jnp.int32),
            pltpu.VMEM((NCH, CHUNK), jnp.int32),
            pltpu.VMEM((NCH, CHUNK), jnp.int32),
            pltpu.VMEM((BPW, EMB), jnp.int32),
            pltpu.VMEM((BPW, EMB_HALF), jnp.float32),
            pltpu.VMEM((BPW, EMB_HALF), jnp.float32),
            pltpu.SemaphoreType.DMA,
        ],
        compiler_params=pltpu.CompilerParams(use_tc_tiling_on_sc=False),
    )


@functools.cache
def _make_gather_u():
    return pl.kernel(
        _gather_u_body,
        out_type=jax.ShapeDtypeStruct((B, EMB), jnp.int32),
        mesh=_mesh(),
        scratch_types=[
            pltpu.VMEM((NCH, CHUNK), jnp.int32),
            pltpu.VMEM((BPW, EMB), jnp.int32),
            pltpu.SemaphoreType.DMA,
        ],
        compiler_params=pltpu.CompilerParams(use_tc_tiling_on_sc=False),
    )


def _unpack(w):
    wu = lax.bitcast_convert_type(w, jnp.uint32)
    lo = lax.bitcast_convert_type((wu & 0xFFFF).astype(jnp.uint16), jnp.bfloat16)
    hi = lax.bitcast_convert_type((wu >> 16).astype(jnp.uint16), jnp.bfloat16)
    return lo, hi


def _mlp_body(u, i, b, c, p, upar, ipar, w1u, w1i, w1b, w1c, w1p, b1, w2, b2, o):
    f32 = jnp.float32
    ulo, uhi = _unpack(u[...])
    ilo, ihi = _unpack(i[...])
    ub = jnp.where(upar[...] != 0, uhi, ulo)
    ib = jnp.where(ipar[...] != 0, ihi, ilo)
    h = jnp.dot(ub, w1u[...], preferred_element_type=f32)
    h = h + jnp.dot(ib, w1i[...], preferred_element_type=f32)
    h = h + jnp.dot(b[...].astype(jnp.bfloat16), w1b[...], preferred_element_type=f32)
    h = h + jnp.dot(c[...].astype(jnp.bfloat16), w1c[...], preferred_element_type=f32)
    h = h + p[...] * w1p[...]
    h = h + b1[...]
    h = jnp.maximum(h, 0.0)
    o[...] = jnp.dot(h, w2[...], preferred_element_type=f32) + b2[...]


def _make_mlp(interpret=False):
    rep = lambda idx: (0, 0)
    row = lambda idx: (idx, 0)
    return pl.pallas_call(
        _mlp_body,
        grid=(B // BLK,),
        in_specs=[
            pl.BlockSpec((BLK, EMB), row),
            pl.BlockSpec((BLK, EMB), row),
            pl.BlockSpec((BLK, EMB_HALF), row),
            pl.BlockSpec((BLK, EMB_HALF), row),
            pl.BlockSpec((BLK, 1), row),
            pl.BlockSpec((BLK, 1), row),
            pl.BlockSpec((BLK, 1), row),
            pl.BlockSpec((EMB, HID), rep),
            pl.BlockSpec((EMB, HID), rep),
            pl.BlockSpec((EMB_HALF, HID), rep),
            pl.BlockSpec((EMB_HALF, HID), rep),
            pl.BlockSpec((1, HID), rep),
            pl.BlockSpec((1, HID), rep),
            pl.BlockSpec((HID, 1), rep),
            pl.BlockSpec((1, 1), rep),
        ],
        out_specs=pl.BlockSpec((BLK, 1), row),
        out_shape=jax.ShapeDtypeStruct((B, 1), jnp.float32),
        interpret=interpret,
    )


_mlp = _make_mlp()


def kernel(user, item, brand, category, price, user_emb, item_emb, brand_emb,
           cat_emb, W1, b1, W2, b2):
    u_idx = _carrier_slot(user.astype(jnp.int32)).reshape(NW, NCH, CHUNK)
    i_idx = _carrier_slot(item.astype(jnp.int32)).reshape(NW, NCH, CHUNK)
    b_idx = brand.astype(jnp.int32).reshape(NW, NCH, CHUNK)
    c_idx = category.astype(jnp.int32).reshape(NW, NCH, CHUNK)

    eye = jnp.eye(EMB, dtype=jnp.bfloat16)
    i_carr = _make_flatten(N_ITEMS)(item_emb.T, eye)
    i_tab = i_carr.reshape(i_carr.shape[0] * 2, EMB)
    i, b, c = _make_gather_ibc()(i_idx, b_idx, c_idx, i_tab, brand_emb, cat_emb)

    u_carr = _make_flatten(N_USERS)(user_emb.T, eye)
    u_tab = u_carr.reshape(u_carr.shape[0] * 2, EMB)
    u = _make_gather_u()(u_idx, u_tab)

    bf = jnp.bfloat16
    w1u = W1[:EMB].astype(bf)
    w1i = W1[EMB : 2 * EMB].astype(bf)
    w1b = W1[2 * EMB : 2 * EMB + EMB_HALF].astype(bf)
    w1c = W1[2 * EMB + EMB_HALF : 3 * EMB].astype(bf)
    w1p = W1[3 * EMB :]

    upar = (user.astype(jnp.int32) & 1)[:, None]
    ipar = (item.astype(jnp.int32) & 1)[:, None]
    out = _mlp(
        u, i, b, c, price[:, None], upar, ipar,
        w1u, w1i, w1b, w1c, w1p,
        b1[None, :], W2, b2[None, :],
    )
    return out[:, 0]
